# Initial kernel scaffold; baseline (speedup 1.0000x reference)
import jax
import jax.numpy as jnp
from jax.experimental import pallas as pl


def kernel(x, edge_index, W1, a_src1, a_dst1, b1, W2, a_src2, a_dst2, b2, Wout, bout):
    raise NotImplementedError("write your pallas kernel here")


# speedup vs baseline: 22.4339x; 22.4339x over previous
import functools

import jax
import jax.numpy as jnp
import numpy as np
from jax import lax
from jax.experimental import pallas as pl
from jax.experimental.pallas import tpu as pltpu
from jax.experimental.pallas import tpu_sc as plsc

N = 10000
E = 320000
D_IN = 128
HID = 32
HEADS = 8
D_OUT = 128

N_PAD = 10240
NE = E + N
C = 128
NT = 32
E_PAD = ((NE + NT * C - 1) // (NT * C)) * (NT * C)
EPT = E_PAD // NT
NA = 10016
ROWS_A = NA // 16
BR = 512

f32 = jnp.float32
i32 = jnp.int32

_cols128 = np.arange(128)[None, :] // HID
_rows16 = np.arange(16)[:, None]
E1A = (_cols128 == _rows16).astype(np.float32)
E1B = (_cols128 + 4 == _rows16).astype(np.float32)
E2 = (np.arange(32)[None, :] * 0 == _rows16).astype(np.float32)


def _embed_attn(a):
    h_, c_ = a.shape
    rows = jnp.arange(h_ * c_)
    return jnp.zeros((h_ * c_, 16), f32).at[rows, rows // c_].set(
        a.reshape(-1).astype(f32))


def _pre1_body(x_ref, w1_ref, as_w, ad_w, ha_ref, hb_ref, as_ref, ad_ref):
    h = jnp.dot(x_ref[...], w1_ref[...], preferred_element_type=f32)
    ha_ref[...] = h[:, :128]
    hb_ref[...] = h[:, 128:]
    as_ref[...] = jnp.dot(h, as_w[...], preferred_element_type=f32)
    ad_ref[...] = jnp.dot(h, ad_w[...], preferred_element_type=f32)


def _pre1(x_p, W1, As, Ad):
    grid = (N_PAD // BR,)
    full = lambda i: (0, 0)
    row = lambda i: (i, 0)
    return pl.pallas_call(
        _pre1_body,
        grid=grid,
        in_specs=[pl.BlockSpec((BR, D_IN), row),
                  pl.BlockSpec((D_IN, 2 * 128), full),
                  pl.BlockSpec((2 * 128, 16), full),
                  pl.BlockSpec((2 * 128, 16), full)],
        out_specs=[pl.BlockSpec((BR, 128), row),
                   pl.BlockSpec((BR, 128), row),
                   pl.BlockSpec((BR, 16), row),
                   pl.BlockSpec((BR, 16), row)],
        out_shape=[jax.ShapeDtypeStruct((N_PAD, 128), f32),
                   jax.ShapeDtypeStruct((N_PAD, 128), f32),
                   jax.ShapeDtypeStruct((N_PAD, 16), f32),
                   jax.ShapeDtypeStruct((N_PAD, 16), f32)],
    )(x_p, W1, As, Ad)


def _elu(u):
    return jnp.where(u > 0, u, jnp.exp(jnp.minimum(u, 0.0)) - 1.0)


def _mid_body(pa_ref, pb_ref, den_ref, w2a_ref, w2b_ref, b1a_ref, b1b_ref,
              e1a_ref, e1b_ref, as_w, ad_w, h2_ref, as_ref, ad_ref):
    den = den_ref[0] + den_ref[1]
    da = jnp.maximum(jnp.dot(den, e1a_ref[...], preferred_element_type=f32), 1e-30)
    db = jnp.maximum(jnp.dot(den, e1b_ref[...], preferred_element_type=f32), 1e-30)
    ua = (pa_ref[0] + pa_ref[1]) / da + b1a_ref[...]
    ub = (pb_ref[0] + pb_ref[1]) / db + b1b_ref[...]
    ha = _elu(ua)
    hb = _elu(ub)
    h2 = (jnp.dot(ha, w2a_ref[...], preferred_element_type=f32)
          + jnp.dot(hb, w2b_ref[...], preferred_element_type=f32))
    h2_ref[...] = h2
    as_ref[...] = jnp.dot(h2, as_w[...], preferred_element_type=f32)
    ad_ref[...] = jnp.dot(h2, ad_w[...], preferred_element_type=f32)


def _mid(pa, pb, den, W2a, W2b, b1a, b1b, As2, Ad2):
    grid = (N_PAD // BR,)
    full = lambda i: (0, 0)
    row3 = lambda i: (0, i, 0)
    row = lambda i: (i, 0)
    return pl.pallas_call(
        _mid_body,
        grid=grid,
        in_specs=[pl.BlockSpec((2, BR, 128), row3),
                  pl.BlockSpec((2, BR, 128), row3),
                  pl.BlockSpec((2, BR, 16), row3),
                  pl.BlockSpec((128, HID), full),
                  pl.BlockSpec((128, HID), full),
                  pl.BlockSpec((1, 128), full),
                  pl.BlockSpec((1, 128), full),
                  pl.BlockSpec((16, 128), full),
                  pl.BlockSpec((16, 128), full),
                  pl.BlockSpec((HID, 16), full),
                  pl.BlockSpec((HID, 16), full)],
        out_specs=[pl.BlockSpec((BR, HID), row),
                   pl.BlockSpec((BR, 16), row),
                   pl.BlockSpec((BR, 16), row)],
        out_shape=[jax.ShapeDtypeStruct((N_PAD, HID), f32),
                   jax.ShapeDtypeStruct((N_PAD, 16), f32),
                   jax.ShapeDtypeStruct((N_PAD, 16), f32)],
    )(pa, pb, den, W2a, W2b, b1a, b1b, E1A, E1B, As2, Ad2)


def _post_body(p_ref, den_ref, e2_ref, b2_ref, wout_ref, bout_ref, y_ref):
    den = den_ref[0] + den_ref[1]
    d = jnp.maximum(jnp.dot(den, e2_ref[...], preferred_element_type=f32), 1e-30)
    u = (p_ref[0] + p_ref[1]) / d + b2_ref[...]
    h = _elu(u)
    y_ref[...] = jnp.dot(h, wout_ref[...], preferred_element_type=f32) + bout_ref[...]


def _post(p2, den2, b2r, Wout, boutr):
    grid = (N_PAD // BR,)
    full = lambda i: (0, 0)
    row3 = lambda i: (0, i, 0)
    row = lambda i: (i, 0)
    return pl.pallas_call(
        _post_body,
        grid=grid,
        in_specs=[pl.BlockSpec((2, BR, HID), row3),
                  pl.BlockSpec((2, BR, 16), row3),
                  pl.BlockSpec((16, HID), full),
                  pl.BlockSpec((1, HID), full),
                  pl.BlockSpec((HID, D_OUT), full),
                  pl.BlockSpec((1, D_OUT), full)],
        out_specs=[pl.BlockSpec((BR, D_OUT), row)],
        out_shape=[jax.ShapeDtypeStruct((N_PAD, D_OUT), f32)],
    )(p2, den2, E2, b2r, Wout, boutr)


def _make_edge(d_feat, head_base, compute_den):
    nvec = d_feat // 16
    mesh = plsc.VectorSubcoreMesh(core_axis_name="c", subcore_axis_name="s")

    out_type = [jax.ShapeDtypeStruct((2, N_PAD, d_feat), f32)]
    scratch = [
        pltpu.VMEM((C,), i32),
        pltpu.VMEM((C,), i32),
        pltpu.VMEM((C, 16), f32),
        pltpu.VMEM((C, 16), f32),
        pltpu.VMEM((C, 16), f32),
        pltpu.VMEM((C, d_feat), f32),
        pltpu.VMEM((C, d_feat), f32),
        pltpu.SemaphoreType.DMA,
        pltpu.SemaphoreType.DMA,
        pltpu.SemaphoreType.DMA,
        pltpu.VMEM_SHARED((NA, d_feat), f32),
    ]
    if compute_den:
        out_type.append(jax.ShapeDtypeStruct((2, N_PAD, 16), f32))
        scratch.append(pltpu.VMEM_SHARED((NA, 16), f32))

    def body(h_hbm, sa_hbm, da_hbm, src_hbm, dst_hbm, out_hbm, *rest):
        if compute_den:
            den_hbm = rest[0]
            rest = rest[1:]
        (src_v, dst_v, sa_v, da_v, w_v, h_v, msg_v, s0, s1, s2, acc) = rest[:11]
        denacc = rest[11] if compute_den else None

        cid = lax.axis_index("c")
        sid = lax.axis_index("s")
        wid = sid * 2 + cid

        @pl.loop(0, C)
        def _(r):
            for j in range(nvec):
                msg_v[r, pl.ds(j * 16, 16)] = jnp.zeros((16,), f32)
            w_v[r, :] = jnp.zeros((16,), f32)

        rb = sid * ROWS_A
        nfull, nrem = ROWS_A // C, ROWS_A % C
        for k in range(nfull):
            pltpu.sync_copy(msg_v, acc.at[pl.ds(rb + k * C, C)])
            if compute_den:
                pltpu.sync_copy(w_v, denacc.at[pl.ds(rb + k * C, C)])
        if nrem:
            pltpu.sync_copy(msg_v.at[pl.ds(0, nrem)],
                            acc.at[pl.ds(rb + nfull * C, nrem)])
            if compute_den:
                pltpu.sync_copy(w_v.at[pl.ds(0, nrem)],
                                denacc.at[pl.ds(rb + nfull * C, nrem)])
        plsc.subcore_barrier()

        ebase = wid * EPT

        @pl.loop(0, EPT, step=C)
        def _(off):
            b = ebase + off
            pltpu.sync_copy(src_hbm.at[pl.ds(b, C)], src_v)
            pltpu.sync_copy(dst_hbm.at[pl.ds(b, C)], dst_v)
            cp0 = pltpu.async_copy(sa_hbm.at[src_v], sa_v, s0)
            cp1 = pltpu.async_copy(da_hbm.at[dst_v], da_v, s1)
            cp2 = pltpu.async_copy(h_hbm.at[src_v], h_v, s2)
            cp0.wait()
            cp1.wait()
            cp2.wait()

            @pl.loop(0, C)
            def _(r):
                e = sa_v[r, :] + da_v[r, :]
                e = jnp.maximum(e, 0.2 * e)
                w = jnp.exp(e)
                w_v[r, :] = w
                for j in range(nvec):
                    lane = head_base + j // 2
                    wj = w.at[jnp.full((16,), lane, i32)].get(
                        mode="promise_in_bounds")
                    msg_v[r, pl.ds(j * 16, 16)] = (
                        h_v[r, pl.ds(j * 16, 16)] * wj)

            pltpu.sync_copy(msg_v, acc.at[dst_v], add=True)
            if compute_den:
                pltpu.sync_copy(w_v, denacc.at[dst_v], add=True)

        plsc.subcore_barrier()
        pltpu.sync_copy(acc.at[pl.ds(rb, ROWS_A)],
                        out_hbm.at[cid, pl.ds(rb, ROWS_A)])
        if compute_den:
            pltpu.sync_copy(denacc.at[pl.ds(rb, ROWS_A)],
                            den_hbm.at[cid, pl.ds(rb, ROWS_A)])

    return pl.kernel(body, out_type=tuple(out_type), mesh=mesh,
                     scratch_types=tuple(scratch),
                     compiler_params=pltpu.CompilerParams(
                         use_tc_tiling_on_sc=False))


_edge1a = _make_edge(128, 0, True)
_edge1b = _make_edge(128, 4, False)
_edge2 = _make_edge(HID, 0, True)


def kernel(x, edge_index, W1, a_src1, a_dst1, b1, W2, a_src2, a_dst2, b2,
           Wout, bout):
    loop = jnp.arange(N, dtype=i32)
    src = jnp.concatenate([edge_index[0].astype(i32), loop])
    dst = jnp.concatenate([edge_index[1].astype(i32), loop])
    npad = E_PAD - NE
    src_p = jnp.concatenate([src, jnp.zeros((npad,), i32)])
    dst_p = jnp.concatenate([dst, jnp.full((npad,), N, i32)])

    x_p = jnp.pad(x.astype(f32), ((0, N_PAD - N), (0, 0)))
    As1 = _embed_attn(a_src1)
    Ad1 = _embed_attn(a_dst1)
    As2 = _embed_attn(a_src2)
    Ad2 = _embed_attn(a_dst2)

    h_a, h_b, asrc1, adst1 = _pre1(x_p, W1.astype(f32), As1, Ad1)

    pa, den1 = _edge1a(h_a, asrc1, adst1, src_p, dst_p)
    (pb,) = _edge1b(h_b, asrc1, adst1, src_p, dst_p)

    h2, asrc2, adst2 = _mid(pa, pb, den1,
                            W2[:128].astype(f32), W2[128:].astype(f32),
                            b1[:128].reshape(1, 128).astype(f32),
                            b1[128:].reshape(1, 128).astype(f32),
                            As2, Ad2)

    p2, den2 = _edge2(h2, asrc2, adst2, src_p, dst_p)

    (y,) = _post(p2, den2, b2.reshape(1, HID).astype(f32),
                 Wout.astype(f32), bout.reshape(1, D_OUT).astype(f32))
    return y[:N]


# speedup vs baseline: 33.3883x; 1.4883x over previous
import functools

import jax
import jax.numpy as jnp
import numpy as np
from jax import lax
from jax.experimental import pallas as pl
from jax.experimental.pallas import tpu as pltpu
from jax.experimental.pallas import tpu_sc as plsc

N = 10000
E = 320000
D_IN = 128
HID = 32
HEADS = 8
D_OUT = 128

N_PAD = 10240
NE = E + N
C = 128
NT = 32
NCH = 82
EPT = NCH * C
E_PAD = NT * EPT
NA = 10016
ROWS_A = NA // 16
BR = 512

f32 = jnp.float32
i32 = jnp.int32

_cols128 = np.arange(128)[None, :] // HID
_rows16 = np.arange(16)[:, None]
E1A = (_cols128 == _rows16).astype(np.float32)
E1B = (_cols128 + 4 == _rows16).astype(np.float32)
E2 = (np.arange(32)[None, :] * 0 == _rows16).astype(np.float32)


def _embed_attn(a):
    h_, c_ = a.shape
    rows = jnp.arange(h_ * c_)
    return jnp.zeros((h_ * c_, 16), f32).at[rows, rows // c_].set(
        a.reshape(-1).astype(f32))


def _pre1_body(x_ref, w1_ref, as_w, ad_w, ha_ref, hb_ref, as_ref, ad_ref):
    h = jnp.dot(x_ref[...], w1_ref[...], preferred_element_type=f32)
    ha_ref[...] = h[:, :128]
    hb_ref[...] = h[:, 128:]
    as_ref[...] = jnp.dot(h, as_w[...], preferred_element_type=f32)
    ad_ref[...] = jnp.dot(h, ad_w[...], preferred_element_type=f32)


def _pre1(x_p, W1, As, Ad):
    grid = (N_PAD // BR,)
    full = lambda i: (0, 0)
    row = lambda i: (i, 0)
    return pl.pallas_call(
        _pre1_body,
        grid=grid,
        in_specs=[pl.BlockSpec((BR, D_IN), row),
                  pl.BlockSpec((D_IN, 2 * 128), full),
                  pl.BlockSpec((2 * 128, 16), full),
                  pl.BlockSpec((2 * 128, 16), full)],
        out_specs=[pl.BlockSpec((BR, 128), row),
                   pl.BlockSpec((BR, 128), row),
                   pl.BlockSpec((BR, 16), row),
                   pl.BlockSpec((BR, 16), row)],
        out_shape=[jax.ShapeDtypeStruct((N_PAD, 128), f32),
                   jax.ShapeDtypeStruct((N_PAD, 128), f32),
                   jax.ShapeDtypeStruct((N_PAD, 16), f32),
                   jax.ShapeDtypeStruct((N_PAD, 16), f32)],
    )(x_p, W1, As, Ad)


def _elu(u):
    return jnp.where(u > 0, u, jnp.exp(jnp.minimum(u, 0.0)) - 1.0)


def _mid_body(pa_ref, pb_ref, den_ref, w2a_ref, w2b_ref, b1a_ref, b1b_ref,
              e1a_ref, e1b_ref, as_w, ad_w, h2_ref, as_ref, ad_ref):
    den = den_ref[0] + den_ref[1]
    da = jnp.maximum(jnp.dot(den, e1a_ref[...], preferred_element_type=f32), 1e-30)
    db = jnp.maximum(jnp.dot(den, e1b_ref[...], preferred_element_type=f32), 1e-30)
    ua = (pa_ref[0] + pa_ref[1]) / da + b1a_ref[...]
    ub = (pb_ref[0] + pb_ref[1]) / db + b1b_ref[...]
    ha = _elu(ua)
    hb = _elu(ub)
    h2 = (jnp.dot(ha, w2a_ref[...], preferred_element_type=f32)
          + jnp.dot(hb, w2b_ref[...], preferred_element_type=f32))
    h2_ref[...] = h2
    as_ref[...] = jnp.dot(h2, as_w[...], preferred_element_type=f32)
    ad_ref[...] = jnp.dot(h2, ad_w[...], preferred_element_type=f32)


def _mid(pa, pb, den, W2a, W2b, b1a, b1b, As2, Ad2):
    grid = (N_PAD // BR,)
    full = lambda i: (0, 0)
    row3 = lambda i: (0, i, 0)
    row = lambda i: (i, 0)
    return pl.pallas_call(
        _mid_body,
        grid=grid,
        in_specs=[pl.BlockSpec((2, BR, 128), row3),
                  pl.BlockSpec((2, BR, 128), row3),
                  pl.BlockSpec((2, BR, 16), row3),
                  pl.BlockSpec((128, HID), full),
                  pl.BlockSpec((128, HID), full),
                  pl.BlockSpec((1, 128), full),
                  pl.BlockSpec((1, 128), full),
                  pl.BlockSpec((16, 128), full),
                  pl.BlockSpec((16, 128), full),
                  pl.BlockSpec((HID, 16), full),
                  pl.BlockSpec((HID, 16), full)],
        out_specs=[pl.BlockSpec((BR, HID), row),
                   pl.BlockSpec((BR, 16), row),
                   pl.BlockSpec((BR, 16), row)],
        out_shape=[jax.ShapeDtypeStruct((N_PAD, HID), f32),
                   jax.ShapeDtypeStruct((N_PAD, 16), f32),
                   jax.ShapeDtypeStruct((N_PAD, 16), f32)],
    )(pa, pb, den, W2a, W2b, b1a, b1b, E1A, E1B, As2, Ad2)


def _post_body(p_ref, den_ref, e2_ref, b2_ref, wout_ref, bout_ref, y_ref):
    den = den_ref[0] + den_ref[1]
    d = jnp.maximum(jnp.dot(den, e2_ref[...], preferred_element_type=f32), 1e-30)
    u = (p_ref[0] + p_ref[1]) / d + b2_ref[...]
    h = _elu(u)
    y_ref[...] = jnp.dot(h, wout_ref[...], preferred_element_type=f32) + bout_ref[...]


def _post(p2, den2, b2r, Wout, boutr):
    grid = (N_PAD // BR,)
    full = lambda i: (0, 0)
    row3 = lambda i: (0, i, 0)
    row = lambda i: (i, 0)
    return pl.pallas_call(
        _post_body,
        grid=grid,
        in_specs=[pl.BlockSpec((2, BR, HID), row3),
                  pl.BlockSpec((2, BR, 16), row3),
                  pl.BlockSpec((16, HID), full),
                  pl.BlockSpec((1, HID), full),
                  pl.BlockSpec((HID, D_OUT), full),
                  pl.BlockSpec((1, D_OUT), full)],
        out_specs=[pl.BlockSpec((BR, D_OUT), row)],
        out_shape=[jax.ShapeDtypeStruct((N_PAD, D_OUT), f32)],
    )(p2, den2, E2, b2r, Wout, boutr)


def _make_edge(d_feat, head_base, compute_den, ck=C):
    nvec = d_feat // 16
    ncht = EPT // ck
    mesh = plsc.VectorSubcoreMesh(core_axis_name="c", subcore_axis_name="s")

    out_type = [jax.ShapeDtypeStruct((2, N_PAD, d_feat), f32)]
    scratch = [
        pltpu.VMEM((ncht, ck), i32),
        pltpu.VMEM((ncht, ck), i32),
    ]
    for _b in range(2):
        scratch += [
            pltpu.VMEM((ck, 16), f32),
            pltpu.VMEM((ck, 16), f32),
            pltpu.VMEM((ck, 16), f32),
            pltpu.VMEM((ck, d_feat), f32),
            pltpu.VMEM((ck, d_feat), f32),
            pltpu.SemaphoreType.DMA,
            pltpu.SemaphoreType.DMA,
        ]
    scratch.append(pltpu.VMEM_SHARED((NA, d_feat), f32))
    if compute_den:
        out_type.append(jax.ShapeDtypeStruct((2, N_PAD, 16), f32))
        scratch.append(pltpu.VMEM_SHARED((NA, 16), f32))

    def body(h_hbm, sa_hbm, da_hbm, src_hbm, dst_hbm, out_hbm, *rest):
        if compute_den:
            den_hbm = rest[0]
            rest = rest[1:]
        idxs, idxd = rest[0], rest[1]
        bufs = [rest[2 + 7 * b:2 + 7 * (b + 1)] for b in range(2)]
        acc = rest[16]
        denacc = rest[17] if compute_den else None

        cid = lax.axis_index("c")
        sid = lax.axis_index("s")
        wid = sid * 2 + cid

        pltpu.sync_copy(src_hbm.at[wid], idxs)
        pltpu.sync_copy(dst_hbm.at[wid], idxd)

        msg0, w0 = bufs[0][4], bufs[0][2]

        @pl.loop(0, ck)
        def _(r):
            for j in range(nvec):
                msg0[r, pl.ds(j * 16, 16)] = jnp.zeros((16,), f32)
            w0[r, :] = jnp.zeros((16,), f32)

        rb = sid * ROWS_A
        nfull, nrem = ROWS_A // ck, ROWS_A % ck
        for k in range(nfull):
            pltpu.sync_copy(msg0, acc.at[pl.ds(rb + k * ck, ck)])
            if compute_den:
                pltpu.sync_copy(w0, denacc.at[pl.ds(rb + k * ck, ck)])
        if nrem:
            pltpu.sync_copy(msg0.at[pl.ds(0, nrem)],
                            acc.at[pl.ds(rb + nfull * ck, nrem)])
            if compute_den:
                pltpu.sync_copy(w0.at[pl.ds(0, nrem)],
                                denacc.at[pl.ds(rb + nfull * ck, nrem)])
        plsc.subcore_barrier()

        def start_gathers(c, buf):
            sa_v, da_v, _, h_v, _, gsem, _ = buf
            pltpu.async_copy(sa_hbm.at[idxs.at[c]], sa_v, gsem)
            pltpu.async_copy(da_hbm.at[idxd.at[c]], da_v, gsem)
            pltpu.async_copy(h_hbm.at[idxs.at[c]], h_v, gsem)

        def wait_gathers(c, buf):
            sa_v, da_v, _, h_v, _, gsem, _ = buf
            pltpu.make_async_copy(sa_hbm.at[idxs.at[c]], sa_v, gsem).wait()
            pltpu.make_async_copy(da_hbm.at[idxd.at[c]], da_v, gsem).wait()
            pltpu.make_async_copy(h_hbm.at[idxs.at[c]], h_v, gsem).wait()

        def compute(c, buf):
            sa_v, da_v, w_v, h_v, msg_v, _, _ = buf

            @pl.loop(0, ck)
            def _(r):
                e = sa_v[r, :] + da_v[r, :]
                e = jnp.maximum(e, 0.2 * e)
                w = jnp.exp(e)
                w_v[r, :] = w
                for j in range(nvec):
                    lane = head_base + j // 2
                    wj = w.at[jnp.full((16,), lane, i32)].get(
                        mode="promise_in_bounds")
                    msg_v[r, pl.ds(j * 16, 16)] = (
                        h_v[r, pl.ds(j * 16, 16)] * wj)

        def start_scatters(c, buf):
            _, _, w_v, _, msg_v, _, ssem = buf
            pltpu.async_copy(msg_v, acc.at[idxd.at[c]], ssem, add=True)
            if compute_den:
                pltpu.async_copy(w_v, denacc.at[idxd.at[c]], ssem, add=True)

        def wait_scatters(c, buf):
            _, _, w_v, _, msg_v, _, ssem = buf
            pltpu.make_async_copy(msg_v, acc.at[idxd.at[c]], ssem).wait()
            if compute_den:
                pltpu.make_async_copy(w_v, denacc.at[idxd.at[c]], ssem).wait()

        start_gathers(0, bufs[0])

        @pl.loop(0, ncht, step=2)
        def _(c):
            start_gathers(c + 1, bufs[1])

            @pl.when(c >= 2)
            def _():
                wait_scatters(c - 2, bufs[0])
            wait_gathers(c, bufs[0])
            compute(c, bufs[0])
            start_scatters(c, bufs[0])

            @pl.when(c + 2 < ncht)
            def _():
                start_gathers(c + 2, bufs[0])

            @pl.when(c >= 2)
            def _():
                wait_scatters(c - 1, bufs[1])
            wait_gathers(c + 1, bufs[1])
            compute(c + 1, bufs[1])
            start_scatters(c + 1, bufs[1])

        wait_scatters(ncht - 2, bufs[0])
        wait_scatters(ncht - 1, bufs[1])

        plsc.subcore_barrier()
        pltpu.sync_copy(acc.at[pl.ds(rb, ROWS_A)],
                        out_hbm.at[cid, pl.ds(rb, ROWS_A)])
        if compute_den:
            pltpu.sync_copy(denacc.at[pl.ds(rb, ROWS_A)],
                            den_hbm.at[cid, pl.ds(rb, ROWS_A)])

    return pl.kernel(body, out_type=tuple(out_type), mesh=mesh,
                     scratch_types=tuple(scratch),
                     compiler_params=pltpu.CompilerParams(
                         use_tc_tiling_on_sc=False))


C1 = 32
_edge1a = _make_edge(128, 0, True, C1)
_edge1b = _make_edge(128, 4, False, C1)
_edge2 = _make_edge(HID, 0, True, C)


def kernel(x, edge_index, W1, a_src1, a_dst1, b1, W2, a_src2, a_dst2, b2,
           Wout, bout):
    loop = jnp.arange(N, dtype=i32)
    src = jnp.concatenate([edge_index[0].astype(i32), loop])
    dst = jnp.concatenate([edge_index[1].astype(i32), loop])
    npad = E_PAD - NE
    src_f = jnp.concatenate([src, jnp.zeros((npad,), i32)])
    dst_f = jnp.concatenate([dst, jnp.full((npad,), N, i32)])
    src_1 = src_f.reshape(NT, EPT // C1, C1)
    dst_1 = dst_f.reshape(NT, EPT // C1, C1)
    src_2 = src_f.reshape(NT, EPT // C, C)
    dst_2 = dst_f.reshape(NT, EPT // C, C)

    x_p = jnp.pad(x.astype(f32), ((0, N_PAD - N), (0, 0)))
    As1 = _embed_attn(a_src1)
    Ad1 = _embed_attn(a_dst1)
    As2 = _embed_attn(a_src2)
    Ad2 = _embed_attn(a_dst2)

    h_a, h_b, asrc1, adst1 = _pre1(x_p, W1.astype(f32), As1, Ad1)

    pa, den1 = _edge1a(h_a, asrc1, adst1, src_1, dst_1)
    (pb,) = _edge1b(h_b, asrc1, adst1, src_1, dst_1)

    h2, asrc2, adst2 = _mid(pa, pb, den1,
                            W2[:128].astype(f32), W2[128:].astype(f32),
                            b1[:128].reshape(1, 128).astype(f32),
                            b1[128:].reshape(1, 128).astype(f32),
                            As2, Ad2)

    p2, den2 = _edge2(h2, asrc2, adst2, src_2, dst_2)

    (y,) = _post(p2, den2, b2.reshape(1, HID).astype(f32),
                 Wout.astype(f32), bout.reshape(1, D_OUT).astype(f32))
    return y[:N]
